# baseline (speedup 1.0000x reference)
import jax
import jax.numpy as jnp
from jax.experimental import pallas as pl
from jax.experimental.pallas import tpu as pltpu

_BM = 512


def _matmul_block(ha_ref, hb_ref, wqa_ref, wqb_ref, out_ref):
    dn = (((1,), (1,)), ((), ()))
    out_ref[...] = jax.lax.dot_general(
        ha_ref[...], wqa_ref[...], dimension_numbers=dn,
        preferred_element_type=jnp.float32,
        precision=jax.lax.Precision.DEFAULT,
    ) + jax.lax.dot_general(
        hb_ref[...], wqb_ref[...], dimension_numbers=dn,
        preferred_element_type=jnp.float32,
        precision=jax.lax.Precision.DEFAULT,
    )


@jax.jit
def kernel(h, Wq, Wn):
    del Wn
    m, d = h.shape
    e = Wq.shape[0]
    grid = (m // _BM,)
    return pl.pallas_call(
        _matmul_block,
        grid=grid,
        in_specs=[
            pl.BlockSpec((_BM, d // 2), lambda i: (i, 0)),
            pl.BlockSpec((_BM, d // 2), lambda i: (i, 1)),
            pl.BlockSpec((e, d // 2), lambda i: (0, 0)),
            pl.BlockSpec((e, d // 2), lambda i: (0, 1)),
        ],
        out_specs=pl.BlockSpec((_BM, e), lambda i: (i, 0)),
        out_shape=jax.ShapeDtypeStruct((m, e), jnp.float32),
        compiler_params=pltpu.CompilerParams(
            dimension_semantics=("arbitrary",),
        ),
    )(h, h, Wq, Wq)


# speedup vs baseline: 1.0019x; 1.0019x over previous
import jax
import jax.numpy as jnp
from jax.experimental import pallas as pl
from jax.experimental.pallas import tpu as pltpu

_BM = 512


def _matmul_block(h_ref, wq_ref, out_ref):
    out_ref[...] = jax.lax.dot_general(
        h_ref[...],
        wq_ref[...],
        dimension_numbers=(((1,), (1,)), ((), ())),
        preferred_element_type=jnp.float32,
        precision=jax.lax.Precision.DEFAULT,
    )


@jax.jit
def kernel(h, Wq, Wn):
    del Wn
    m, d = h.shape
    e = Wq.shape[0]
    grid = (m // _BM,)
    return pl.pallas_call(
        _matmul_block,
        grid=grid,
        in_specs=[
            pl.BlockSpec((_BM, d), lambda i: (i, 0)),
            pl.BlockSpec((e, d), lambda i: (0, 0)),
        ],
        out_specs=pl.BlockSpec((_BM, e), lambda i: (i, 0)),
        out_shape=jax.ShapeDtypeStruct((m, e), jnp.float32),
        compiler_params=pltpu.CompilerParams(
            dimension_semantics=("arbitrary",),
            needs_layout_passes=True,
        ),
    )(h, Wq)
